# baseline (speedup 1.0000x reference)
import functools

import jax
import jax.numpy as jnp
from jax import lax
from jax.experimental import pallas as pl
from jax.experimental.pallas import tpu as pltpu
from jax.experimental.pallas import tpu_sc as plsc

N = 10000
E = 320000
D = 128
NHID = 128
N_CLASSES = 64
NUM_GRAPHS = 128

N_PAD = 10240
E_PAD = 331776
CHUNK = 64
DCHUNK = 128
NCHUNKS = E_PAD // 16 // DCHUNK
CHUNKS_PER_TILE = E_PAD // 32 // CHUNK
STAGES = 3
SCHUNKS = CHUNKS_PER_TILE // STAGES

_MESH = plsc.VectorSubcoreMesh(core_axis_name="c", subcore_axis_name="s")
_SC_PARAMS = pltpu.CompilerParams(needs_layout_passes=False)


def _rsqrt_newton(d):
    i = lax.bitcast_convert_type(d, jnp.int32)
    y = lax.bitcast_convert_type(jnp.int32(0x5F3759DF) - (i >> 1), jnp.float32)
    for _ in range(3):
        y = y * (1.5 - 0.5 * d * y * y)
    return y


@functools.partial(
    pl.kernel,
    out_type=jax.ShapeDtypeStruct((32, NCHUNKS // 2, DCHUNK), jnp.float32),
    mesh=_MESH,
    compiler_params=_SC_PARAMS,
    scratch_types=[
        pltpu.VMEM((NCHUNKS, DCHUNK), jnp.int32),
        pltpu.VMEM((NCHUNKS, DCHUNK), jnp.int32),
        pltpu.VMEM((NCHUNKS, DCHUNK), jnp.float32),
        pltpu.VMEM((N_PAD,), jnp.float32),
        pltpu.VMEM((640,), jnp.float32),
        pltpu.VMEM_SHARED((N_PAD,), jnp.float32),
        pltpu.SemaphoreType.DMA,
    ],
)
def _norm_kernel(row_h, col_h, ew_h, norm_h,
                 row_v, col_v, ew_v, dis_v, nbuf, deg_sh, dsem):
    c = lax.axis_index("c")
    s = lax.axis_index("s")

    def zero16(i, _):
        nbuf[pl.ds(i * 16, 16)] = jnp.zeros((16,), jnp.float32)
        return 0

    lax.fori_loop(0, 40, zero16, 0)
    pltpu.sync_copy(nbuf, deg_sh.at[pl.ds(s * 640, 640)])

    pltpu.sync_copy(row_h.at[s], row_v)
    pltpu.sync_copy(col_h.at[s], col_v)
    pltpu.sync_copy(ew_h.at[s], ew_v)
    plsc.subcore_barrier()

    def scat8(g, _):
        for i in range(8):
            pltpu.async_copy(ew_v.at[g * 8 + i], deg_sh.at[col_v.at[g * 8 + i]],
                             dsem, add=True)
        for i in range(8):
            pltpu.make_async_copy(ew_v.at[0], deg_sh.at[col_v.at[0]],
                                  dsem).wait()
        return 0

    lax.fori_loop(0, NCHUNKS // 8, scat8, 0)
    for j in range(NCHUNKS - NCHUNKS % 8, NCHUNKS):
        pltpu.sync_copy(ew_v.at[j], deg_sh.at[col_v.at[j]], add=True)
    plsc.subcore_barrier()

    pltpu.sync_copy(deg_sh.at[pl.ds(s * 640, 640)], nbuf)

    def newton(i, _):
        nbuf[pl.ds(i * 16, 16)] = _rsqrt_newton(nbuf[pl.ds(i * 16, 16)])
        return 0

    lax.fori_loop(0, 40, newton, 0)
    pltpu.sync_copy(nbuf, deg_sh.at[pl.ds(s * 640, 640)])
    plsc.subcore_barrier()

    pltpu.sync_copy(deg_sh, dis_v)

    half = NCHUNKS // 2

    def nrow(r, _):
        rr = c * half + r
        for k in range(DCHUNK // 16):
            sl = pl.ds(k * 16, 16)
            a = plsc.load_gather(dis_v, [row_v[rr, sl]])
            b2 = plsc.load_gather(dis_v, [col_v[rr, sl]])
            ew_v[rr, sl] = a * ew_v[rr, sl] * b2
        return 0

    lax.fori_loop(0, half, nrow, 0, unroll=2)
    pltpu.sync_copy(ew_v.at[pl.ds(c * half, half)], norm_h.at[s * 2 + c])


def _make_hop():
    @functools.partial(
        pl.kernel,
        out_type=jax.ShapeDtypeStruct((2, N_PAD, D), jnp.float32),
        mesh=_MESH,
        compiler_params=_SC_PARAMS,
        scratch_types=[
            pltpu.VMEM((SCHUNKS, CHUNK), jnp.int32),
            pltpu.VMEM((SCHUNKS, CHUNK), jnp.int32),
            pltpu.VMEM((SCHUNKS, CHUNK), jnp.float32),
            pltpu.VMEM((CHUNK, D), jnp.float32),
            pltpu.VMEM((CHUNK, D), jnp.float32),
            pltpu.VMEM((CHUNK, D), jnp.float32),
            pltpu.VMEM_SHARED((N_PAD, D), jnp.float32),
            pltpu.SemaphoreType.DMA,
            pltpu.SemaphoreType.DMA,
            pltpu.SemaphoreType.DMA,
            pltpu.SemaphoreType.DMA,
            pltpu.SemaphoreType.DMA,
            pltpu.SemaphoreType.DMA,
        ],
    )
    def _hop(row_h, col_h, norm_h, h_h, out_h,
             row_v, col_v, norm_v, rows_a, rows_b, rows_c, acc_sh,
             gsa, gsb, gsc, ssa, ssb, ssc):
        c = lax.axis_index("c")
        s = lax.axis_index("s")
        wid = s * 2 + c

        def zrow(r, _):
            for k in range(D // 16):
                rows_a[r, pl.ds(k * 16, 16)] = jnp.zeros((16,), jnp.float32)
            return 0

        lax.fori_loop(0, CHUNK, zrow, 0)
        for b in range(640 // CHUNK):
            pltpu.async_copy(rows_a,
                             acc_sh.at[pl.ds(s * 640 + b * CHUNK, CHUNK)], gsa)
        for b in range(640 // CHUNK):
            pltpu.make_async_copy(rows_a, acc_sh.at[pl.ds(0, CHUNK)],
                                  gsa).wait()
        plsc.subcore_barrier()

        def gather(j, buf, sem):
            pltpu.async_copy(h_h.at[row_v.at[j]], buf, sem)

        def gather_wait(buf, sem):
            pltpu.make_async_copy(h_h.at[row_v.at[0]], buf, sem).wait()

        def scat(j, buf, sem):
            pltpu.async_copy(buf, acc_sh.at[col_v.at[j]], sem, add=True)

        def scat_wait(buf, sem):
            pltpu.make_async_copy(buf, acc_sh.at[col_v.at[0]], sem).wait()

        def scale(buf, j):
            def scale_row(r, _):
                sc = plsc.load_gather(norm_v.at[j],
                                      [jnp.full((16,), r, jnp.int32)])
                for k in range(D // 16):
                    sl = pl.ds(k * 16, 16)
                    buf[r, sl] = buf[r, sl] * sc
                return 0

            lax.fori_loop(0, CHUNK, scale_row, 0, unroll=4)

        bufs = (rows_a, rows_b, rows_c)
        gsems = (gsa, gsb, gsc)
        ssems = (ssa, ssb, ssc)

        for st in range(STAGES):
            pltpu.sync_copy(row_h.at[wid, st], row_v)
            pltpu.sync_copy(col_h.at[wid, st], col_v)
            pltpu.sync_copy(norm_h.at[wid, st], norm_v)

            gather(0, rows_a, gsa)
            gather(1, rows_b, gsb)

            def triple(g, _):
                j0 = g * 3
                for i in range(3):
                    j = j0 + i
                    buf = bufs[i]
                    nbuf_i = (i + 2) % 3
                    nxt_buf, ngs, nss = bufs[nbuf_i], gsems[nbuf_i], ssems[nbuf_i]
                    gather_wait(buf, gsems[i])
                    scale(buf, j)
                    scat(j, buf, ssems[i])
                    nxt = j + 2

                    @pl.when(nxt < SCHUNKS)
                    def _pf():
                        @pl.when(nxt >= 3)
                        def _ws():
                            scat_wait(nxt_buf, nss)

                        gather(nxt, nxt_buf, ngs)
                return 0

            lax.fori_loop(0, SCHUNKS // 3, triple, 0)
            scat_wait(rows_a, ssa)
            scat_wait(rows_b, ssb)
            scat_wait(rows_c, ssc)
        plsc.subcore_barrier()
        pltpu.sync_copy(acc_sh.at[pl.ds(s * 640, 640)],
                        out_h.at[c, pl.ds(s * 640, 640)])

    return _hop


_hop_kernel = _make_hop()


def _combine_body(p_ref, o_ref):
    o_ref[...] = p_ref[0] + p_ref[1]


def _combine(parts):
    return pl.pallas_call(
        _combine_body,
        grid=(8,),
        in_specs=[pl.BlockSpec((2, N_PAD // 8, D), lambda i: (0, i, 0))],
        out_specs=pl.BlockSpec((N_PAD // 8, D), lambda i: (i, 0)),
        out_shape=jax.ShapeDtypeStruct((N_PAD, D), jnp.float32),
    )(parts)


def _tail_body(p_ref, bt_ref, wc_ref, bc_ref, w1_ref, b1_ref, w2_ref, b2_ref,
               o_ref, sm_ref, cn_ref):
    i = pl.program_id(0)

    @pl.when(i == 0)
    def _init():
        sm_ref[...] = jnp.zeros_like(sm_ref)
        cn_ref[...] = jnp.zeros_like(cn_ref)

    h2 = p_ref[0] + p_ref[1]
    bt = bt_ref[0]
    gid = lax.broadcasted_iota(jnp.int32, (NUM_GRAPHS, 128), 0).astype(
        jnp.float32)
    m = (gid == bt).astype(jnp.float32)
    sm_ref[...] += jnp.dot(m, h2, preferred_element_type=jnp.float32,
                           precision=lax.Precision.HIGHEST)
    cn_ref[...] += jnp.dot(m, jnp.ones((128, 128), jnp.float32),
                           preferred_element_type=jnp.float32,
                           precision=lax.Precision.HIGHEST)

    @pl.when(i == N_PAD // 128 - 1)
    def _finish():
        cn1 = cn_ref[:, 0:1]
        sums = jnp.dot(sm_ref[...], wc_ref[...],
                       preferred_element_type=jnp.float32,
                       precision=lax.Precision.HIGHEST) + cn1 * bc_ref[...]
        pooled = sums / jnp.maximum(cn1, 1.0)
        z = jnp.maximum(jnp.dot(pooled, w1_ref[...],
                                preferred_element_type=jnp.float32,
                                precision=lax.Precision.HIGHEST) + b1_ref[...],
                        0.0)
        z2 = jnp.dot(z, w2_ref[...], preferred_element_type=jnp.float32,
                     precision=lax.Precision.HIGHEST) + b2_ref[...]
        mx = jnp.max(z2, axis=1, keepdims=True)
        o_ref[...] = z2 - mx - jnp.log(jnp.sum(jnp.exp(z2 - mx), axis=1,
                                               keepdims=True))


def _tail(parts, batch3, wcT, bc2, w1T, b12, w2T, b22):
    nblk = N_PAD // 128
    full = lambda i: (0, 0)
    return pl.pallas_call(
        _tail_body,
        grid=(nblk,),
        in_specs=[
            pl.BlockSpec((2, 128, D), lambda i: (0, i, 0)),
            pl.BlockSpec((1, 1, 128), lambda i: (i, 0, 0)),
            pl.BlockSpec((D, NHID), full),
            pl.BlockSpec((1, NHID), full),
            pl.BlockSpec((NHID, NHID), full),
            pl.BlockSpec((1, NHID), full),
            pl.BlockSpec((NHID, N_CLASSES), full),
            pl.BlockSpec((1, N_CLASSES), full),
        ],
        out_specs=pl.BlockSpec((NUM_GRAPHS, N_CLASSES), full),
        out_shape=jax.ShapeDtypeStruct((NUM_GRAPHS, N_CLASSES), jnp.float32),
        scratch_shapes=[
            pltpu.VMEM((NUM_GRAPHS, 128), jnp.float32),
            pltpu.VMEM((NUM_GRAPHS, 128), jnp.float32),
        ],
    )(parts, batch3, wcT, bc2, w1T, b12, w2T, b22)


def kernel(x, edge_index, edge_weight, batch, W_conv, b_conv, W1, b1, W2, b2):
    row = edge_index[0].astype(jnp.int32)
    col = edge_index[1].astype(jnp.int32)
    loop = jnp.arange(N, dtype=jnp.int32)
    npad = E_PAD - E - N
    pad_ids = jnp.arange(npad, dtype=jnp.int32)
    row_p = jnp.concatenate([row, loop, pad_ids])
    col_p = jnp.concatenate([col, loop, pad_ids])
    ew_p = jnp.concatenate([edge_weight.astype(jnp.float32),
                            jnp.ones((N,), jnp.float32),
                            jnp.zeros((npad,), jnp.float32)])
    row3 = row_p.reshape(32, STAGES, SCHUNKS, CHUNK)
    col3 = col_p.reshape(32, STAGES, SCHUNKS, CHUNK)
    rowD = row_p.reshape(16, NCHUNKS, DCHUNK)
    colD = col_p.reshape(16, NCHUNKS, DCHUNK)
    ewD = ew_p.reshape(16, NCHUNKS, DCHUNK)

    norm = _norm_kernel(rowD, colD, ewD)
    norm3 = norm.reshape(32, STAGES, SCHUNKS, CHUNK)
    p1 = _hop_kernel(row3, col3, norm3, x)
    h1 = _combine(p1)
    p2 = _hop_kernel(row3, col3, norm3, h1)

    batch_pad = jnp.concatenate([batch.astype(jnp.float32),
                                 jnp.full((N_PAD - N,), 200.0, jnp.float32)])
    batch3 = batch_pad.reshape(N_PAD // 128, 1, 128)
    out = _tail(p2, batch3, W_conv.T, b_conv.reshape(1, NHID),
                W1.T, b1.reshape(1, NHID), W2.T, b2.reshape(1, N_CLASSES))
    return out


# speedup vs baseline: 1.0214x; 1.0214x over previous
import functools

import jax
import jax.numpy as jnp
from jax import lax
from jax.experimental import pallas as pl
from jax.experimental.pallas import tpu as pltpu
from jax.experimental.pallas import tpu_sc as plsc

N = 10000
E = 320000
D = 128
NHID = 128
N_CLASSES = 64
NUM_GRAPHS = 128

N_PAD = 10240
E_PAD = 331776
CHUNK = 64
DCHUNK = 128
NCHUNKS = E_PAD // 16 // DCHUNK
CHUNKS_PER_TILE = E_PAD // 32 // CHUNK
STAGES = 3
SCHUNKS = CHUNKS_PER_TILE // STAGES

_MESH = plsc.VectorSubcoreMesh(core_axis_name="c", subcore_axis_name="s")
_SC_PARAMS = pltpu.CompilerParams(needs_layout_passes=False)


def _rsqrt_newton(d):
    i = lax.bitcast_convert_type(d, jnp.int32)
    y = lax.bitcast_convert_type(jnp.int32(0x5F3759DF) - (i >> 1), jnp.float32)
    for _ in range(3):
        y = y * (1.5 - 0.5 * d * y * y)
    return y


@functools.partial(
    pl.kernel,
    out_type=jax.ShapeDtypeStruct((32, NCHUNKS // 2, DCHUNK), jnp.float32),
    mesh=_MESH,
    compiler_params=_SC_PARAMS,
    scratch_types=[
        pltpu.VMEM((NCHUNKS, DCHUNK), jnp.int32),
        pltpu.VMEM((NCHUNKS, DCHUNK), jnp.int32),
        pltpu.VMEM((NCHUNKS, DCHUNK), jnp.float32),
        pltpu.VMEM((N_PAD,), jnp.float32),
        pltpu.VMEM((640,), jnp.float32),
        pltpu.VMEM_SHARED((N_PAD,), jnp.float32),
        pltpu.SemaphoreType.DMA,
    ],
)
def _norm_kernel(row_h, col_h, ew_h, norm_h,
                 row_v, col_v, ew_v, dis_v, nbuf, deg_sh, dsem):
    c = lax.axis_index("c")
    s = lax.axis_index("s")

    def zero16(i, _):
        nbuf[pl.ds(i * 16, 16)] = jnp.zeros((16,), jnp.float32)
        return 0

    lax.fori_loop(0, 40, zero16, 0)
    pltpu.sync_copy(nbuf, deg_sh.at[pl.ds(s * 640, 640)])

    pltpu.async_copy(row_h.at[s], row_v, dsem)
    pltpu.async_copy(col_h.at[s], col_v, dsem)
    pltpu.async_copy(ew_h.at[s], ew_v, dsem)
    pltpu.make_async_copy(row_h.at[s], row_v, dsem).wait()
    pltpu.make_async_copy(col_h.at[s], col_v, dsem).wait()
    pltpu.make_async_copy(ew_h.at[s], ew_v, dsem).wait()
    plsc.subcore_barrier()

    def scat8(g, _):
        for i in range(8):
            pltpu.async_copy(ew_v.at[g * 8 + i], deg_sh.at[col_v.at[g * 8 + i]],
                             dsem, add=True)
        for i in range(8):
            pltpu.make_async_copy(ew_v.at[0], deg_sh.at[col_v.at[0]],
                                  dsem).wait()
        return 0

    lax.fori_loop(0, NCHUNKS // 8, scat8, 0)
    for j in range(NCHUNKS - NCHUNKS % 8, NCHUNKS):
        pltpu.sync_copy(ew_v.at[j], deg_sh.at[col_v.at[j]], add=True)
    plsc.subcore_barrier()

    pltpu.sync_copy(deg_sh.at[pl.ds(s * 640, 640)], nbuf)

    def newton(i, _):
        nbuf[pl.ds(i * 16, 16)] = _rsqrt_newton(nbuf[pl.ds(i * 16, 16)])
        return 0

    lax.fori_loop(0, 40, newton, 0)
    pltpu.sync_copy(nbuf, deg_sh.at[pl.ds(s * 640, 640)])
    plsc.subcore_barrier()

    pltpu.sync_copy(deg_sh, dis_v)

    half = NCHUNKS // 2

    def nrow(r, _):
        rr = c * half + r
        for k in range(DCHUNK // 16):
            sl = pl.ds(k * 16, 16)
            a = plsc.load_gather(dis_v, [row_v[rr, sl]])
            b2 = plsc.load_gather(dis_v, [col_v[rr, sl]])
            ew_v[rr, sl] = a * ew_v[rr, sl] * b2
        return 0

    lax.fori_loop(0, half, nrow, 0, unroll=2)
    pltpu.sync_copy(ew_v.at[pl.ds(c * half, half)], norm_h.at[s * 2 + c])


def _make_hop():
    @functools.partial(
        pl.kernel,
        out_type=jax.ShapeDtypeStruct((2, N_PAD, D), jnp.float32),
        mesh=_MESH,
        compiler_params=_SC_PARAMS,
        scratch_types=[
            pltpu.VMEM((SCHUNKS, CHUNK), jnp.int32),
            pltpu.VMEM((SCHUNKS, CHUNK), jnp.int32),
            pltpu.VMEM((SCHUNKS, CHUNK), jnp.float32),
            pltpu.VMEM((CHUNK, D), jnp.float32),
            pltpu.VMEM((CHUNK, D), jnp.float32),
            pltpu.VMEM((CHUNK, D), jnp.float32),
            pltpu.VMEM_SHARED((N_PAD, D), jnp.float32),
            pltpu.SemaphoreType.DMA,
            pltpu.SemaphoreType.DMA,
            pltpu.SemaphoreType.DMA,
            pltpu.SemaphoreType.DMA,
            pltpu.SemaphoreType.DMA,
            pltpu.SemaphoreType.DMA,
        ],
    )
    def _hop(row_h, col_h, norm_h, h_h, out_h,
             row_v, col_v, norm_v, rows_a, rows_b, rows_c, acc_sh,
             gsa, gsb, gsc, ssa, ssb, ssc):
        c = lax.axis_index("c")
        s = lax.axis_index("s")
        wid = s * 2 + c

        def zrow(r, _):
            for k in range(D // 16):
                rows_a[r, pl.ds(k * 16, 16)] = jnp.zeros((16,), jnp.float32)
            return 0

        lax.fori_loop(0, CHUNK, zrow, 0)
        for b in range(640 // CHUNK):
            pltpu.async_copy(rows_a,
                             acc_sh.at[pl.ds(s * 640 + b * CHUNK, CHUNK)], gsa)
        for b in range(640 // CHUNK):
            pltpu.make_async_copy(rows_a, acc_sh.at[pl.ds(0, CHUNK)],
                                  gsa).wait()
        plsc.subcore_barrier()

        def gather(j, buf, sem):
            pltpu.async_copy(h_h.at[row_v.at[j]], buf, sem)

        def gather_wait(buf, sem):
            pltpu.make_async_copy(h_h.at[row_v.at[0]], buf, sem).wait()

        def scat(j, buf, sem):
            pltpu.async_copy(buf, acc_sh.at[col_v.at[j]], sem, add=True)

        def scat_wait(buf, sem):
            pltpu.make_async_copy(buf, acc_sh.at[col_v.at[0]], sem).wait()

        def scale(buf, j):
            def scale_row(r, _):
                sc = plsc.load_gather(norm_v.at[j],
                                      [jnp.full((16,), r, jnp.int32)])
                for k in range(D // 16):
                    sl = pl.ds(k * 16, 16)
                    buf[r, sl] = buf[r, sl] * sc
                return 0

            lax.fori_loop(0, CHUNK, scale_row, 0, unroll=4)

        bufs = (rows_a, rows_b, rows_c)
        gsems = (gsa, gsb, gsc)
        ssems = (ssa, ssb, ssc)

        for st in range(STAGES):
            pltpu.async_copy(row_h.at[wid, st], row_v, gsa)
            pltpu.async_copy(col_h.at[wid, st], col_v, gsa)
            pltpu.async_copy(norm_h.at[wid, st], norm_v, gsa)
            pltpu.make_async_copy(row_h.at[wid, st], row_v, gsa).wait()
            pltpu.make_async_copy(col_h.at[wid, st], col_v, gsa).wait()
            pltpu.make_async_copy(norm_h.at[wid, st], norm_v, gsa).wait()

            gather(0, rows_a, gsa)
            gather(1, rows_b, gsb)

            def triple(g, _):
                j0 = g * 3
                for i in range(3):
                    j = j0 + i
                    buf = bufs[i]
                    nbuf_i = (i + 2) % 3
                    nxt_buf, ngs, nss = bufs[nbuf_i], gsems[nbuf_i], ssems[nbuf_i]
                    gather_wait(buf, gsems[i])
                    scale(buf, j)
                    scat(j, buf, ssems[i])
                    nxt = j + 2

                    @pl.when(nxt < SCHUNKS)
                    def _pf():
                        @pl.when(nxt >= 3)
                        def _ws():
                            scat_wait(nxt_buf, nss)

                        gather(nxt, nxt_buf, ngs)
                return 0

            lax.fori_loop(0, SCHUNKS // 3, triple, 0)
            scat_wait(rows_a, ssa)
            scat_wait(rows_b, ssb)
            scat_wait(rows_c, ssc)
        plsc.subcore_barrier()
        pltpu.sync_copy(acc_sh.at[pl.ds(s * 640, 640)],
                        out_h.at[c, pl.ds(s * 640, 640)])

    return _hop


_hop_kernel = _make_hop()


def _combine_body(p_ref, o_ref):
    o_ref[...] = p_ref[0] + p_ref[1]


def _combine(parts):
    return pl.pallas_call(
        _combine_body,
        grid=(8,),
        in_specs=[pl.BlockSpec((2, N_PAD // 8, D), lambda i: (0, i, 0))],
        out_specs=pl.BlockSpec((N_PAD // 8, D), lambda i: (i, 0)),
        out_shape=jax.ShapeDtypeStruct((N_PAD, D), jnp.float32),
    )(parts)


def _tail_body(p_ref, bt_ref, wc_ref, bc_ref, w1_ref, b1_ref, w2_ref, b2_ref,
               o_ref, sm_ref, cn_ref):
    i = pl.program_id(0)

    @pl.when(i == 0)
    def _init():
        sm_ref[...] = jnp.zeros_like(sm_ref)
        cn_ref[...] = jnp.zeros_like(cn_ref)

    h2 = p_ref[0] + p_ref[1]
    bt = bt_ref[0]
    gid = lax.broadcasted_iota(jnp.int32, (NUM_GRAPHS, 128), 0).astype(
        jnp.float32)
    m = (gid == bt).astype(jnp.float32)
    sm_ref[...] += jnp.dot(m, h2, preferred_element_type=jnp.float32,
                           precision=lax.Precision.HIGHEST)
    cn_ref[...] += jnp.dot(m, jnp.ones((128, 128), jnp.float32),
                           preferred_element_type=jnp.float32,
                           precision=lax.Precision.HIGHEST)

    @pl.when(i == N_PAD // 128 - 1)
    def _finish():
        cn1 = cn_ref[:, 0:1]
        sums = jnp.dot(sm_ref[...], wc_ref[...],
                       preferred_element_type=jnp.float32,
                       precision=lax.Precision.HIGHEST) + cn1 * bc_ref[...]
        pooled = sums / jnp.maximum(cn1, 1.0)
        z = jnp.maximum(jnp.dot(pooled, w1_ref[...],
                                preferred_element_type=jnp.float32,
                                precision=lax.Precision.HIGHEST) + b1_ref[...],
                        0.0)
        z2 = jnp.dot(z, w2_ref[...], preferred_element_type=jnp.float32,
                     precision=lax.Precision.HIGHEST) + b2_ref[...]
        mx = jnp.max(z2, axis=1, keepdims=True)
        o_ref[...] = z2 - mx - jnp.log(jnp.sum(jnp.exp(z2 - mx), axis=1,
                                               keepdims=True))


def _tail(parts, batch3, wcT, bc2, w1T, b12, w2T, b22):
    nblk = N_PAD // 128
    full = lambda i: (0, 0)
    return pl.pallas_call(
        _tail_body,
        grid=(nblk,),
        in_specs=[
            pl.BlockSpec((2, 128, D), lambda i: (0, i, 0)),
            pl.BlockSpec((1, 1, 128), lambda i: (i, 0, 0)),
            pl.BlockSpec((D, NHID), full),
            pl.BlockSpec((1, NHID), full),
            pl.BlockSpec((NHID, NHID), full),
            pl.BlockSpec((1, NHID), full),
            pl.BlockSpec((NHID, N_CLASSES), full),
            pl.BlockSpec((1, N_CLASSES), full),
        ],
        out_specs=pl.BlockSpec((NUM_GRAPHS, N_CLASSES), full),
        out_shape=jax.ShapeDtypeStruct((NUM_GRAPHS, N_CLASSES), jnp.float32),
        scratch_shapes=[
            pltpu.VMEM((NUM_GRAPHS, 128), jnp.float32),
            pltpu.VMEM((NUM_GRAPHS, 128), jnp.float32),
        ],
    )(parts, batch3, wcT, bc2, w1T, b12, w2T, b22)


def kernel(x, edge_index, edge_weight, batch, W_conv, b_conv, W1, b1, W2, b2):
    row = edge_index[0].astype(jnp.int32)
    col = edge_index[1].astype(jnp.int32)
    loop = jnp.arange(N, dtype=jnp.int32)
    npad = E_PAD - E - N
    pad_ids = jnp.arange(npad, dtype=jnp.int32)
    row_p = jnp.concatenate([row, loop, pad_ids])
    col_p = jnp.concatenate([col, loop, pad_ids])
    ew_p = jnp.concatenate([edge_weight.astype(jnp.float32),
                            jnp.ones((N,), jnp.float32),
                            jnp.zeros((npad,), jnp.float32)])
    row3 = row_p.reshape(32, STAGES, SCHUNKS, CHUNK)
    col3 = col_p.reshape(32, STAGES, SCHUNKS, CHUNK)
    rowD = row_p.reshape(16, NCHUNKS, DCHUNK)
    colD = col_p.reshape(16, NCHUNKS, DCHUNK)
    ewD = ew_p.reshape(16, NCHUNKS, DCHUNK)

    norm = _norm_kernel(rowD, colD, ewD)
    norm3 = norm.reshape(32, STAGES, SCHUNKS, CHUNK)
    p1 = _hop_kernel(row3, col3, norm3, x)
    h1 = _combine(p1)
    p2 = _hop_kernel(row3, col3, norm3, h1)

    batch_pad = jnp.concatenate([batch.astype(jnp.float32),
                                 jnp.full((N_PAD - N,), 200.0, jnp.float32)])
    batch3 = batch_pad.reshape(N_PAD // 128, 1, 128)
    out = _tail(p2, batch3, W_conv.T, b_conv.reshape(1, NHID),
                W1.T, b1.reshape(1, NHID), W2.T, b2.reshape(1, N_CLASSES))
    return out
